# baseline (speedup 1.0000x reference)
import jax
import jax.numpy as jnp
from jax.experimental import pallas as pl
from jax.experimental.pallas import tpu as pltpu

_NS = 3


def _expert_apply(f, w, b):
    off = jnp.dot(
        w.astype(jnp.bfloat16),
        f.astype(jnp.bfloat16),
        preferred_element_type=jnp.float32,
    )
    off = jnp.maximum(off + b, 0.0)
    key = f + off
    mx = jnp.max(key, axis=-1, keepdims=True)
    e = jnp.exp(key - mx)
    return e / jnp.sum(e, axis=-1, keepdims=True)


def _route_kernel(cls_ref, out_ref):
    x = cls_ref[...]
    nb = x.shape[0]
    m = jnp.max(x, axis=-1, keepdims=True)
    iota = jax.lax.broadcasted_iota(jnp.int32, x.shape, 1)
    big = jnp.int32(x.shape[1])
    idx = jnp.min(jnp.where(x == m, iota, big), axis=-1)
    ii = jax.lax.broadcasted_iota(jnp.int32, (nb, nb), 0)
    jj = jax.lax.broadcasted_iota(jnp.int32, (nb, nb), 1)
    cat_row = jnp.broadcast_to(idx[None, :], (nb, nb))
    cat_col = jnp.broadcast_to(idx[:, None], (nb, nb))
    less = (cat_row < cat_col) | ((cat_row == cat_col) & (jj < ii))
    rank = jnp.sum(less.astype(jnp.int32), axis=1)
    eqm = jnp.broadcast_to(rank[None, :], (nb, nb)) == ii
    perm = jnp.sum(jnp.where(eqm, jj, 0), axis=1)
    cats = jnp.sum(jnp.where(eqm, cat_row, 0), axis=1)
    cats_row = jnp.broadcast_to(cats[None, :], (nb, nb))
    prev = jnp.sum(jnp.where(jj == ii - 1, cats_row, 0), axis=1)
    kpos = jax.lax.broadcasted_iota(jnp.int32, (nb,), 0)
    need = ((cats != prev) | (kpos == 0)).astype(jnp.int32)
    need_row = jnp.broadcast_to(need[None, :], (nb, nb))
    dcnt = jnp.sum(jnp.where(jj <= ii, need_row, 0), axis=1) - 1
    nd = jnp.max(dcnt) + 1
    sel = (jnp.broadcast_to(dcnt[None, :], (nb, nb)) == ii) & (need_row == 1)
    dcats = jnp.sum(jnp.where(sel, cats_row, 0), axis=1)
    out_ref[0, :] = idx
    out_ref[1, :] = cats
    out_ref[2, :] = perm
    out_ref[3, :] = need
    out_ref[4, :] = dcnt
    out_ref[5, :] = dcats
    out_ref[6, :] = jnp.broadcast_to(nd, (nb,))
    out_ref[7, :] = idx


def _small_kernel(sp_ref, f2_ref, w2_ref, b2_ref, f3_ref, w3_ref, b3_ref,
                  o2_ref, o3_ref):
    nb = f2_ref.shape[0]
    for b in range(nb):
        c = sp_ref[0, b]
        o2_ref[b] = _expert_apply(f2_ref[b], w2_ref[c], b2_ref[c])
        o3_ref[b] = _expert_apply(f3_ref[b], w3_ref[c], b3_ref[c])


def _big_kernel(sp_ref, feat_hbm, w_hbm, b_ref, out_hbm,
                w_buf, f_res, wsem, fsem, osem):
    nb = f_res.shape[0]
    nd = sp_ref[6, 0]

    def w_copy(d):
        c = sp_ref[5, d]
        return pltpu.make_async_copy(w_hbm.at[c], w_buf.at[d % _NS],
                                     wsem.at[d % _NS])

    f_all = pltpu.make_async_copy(feat_hbm, f_res, fsem)
    f_all.start()
    w_copy(0).start()

    @pl.when(nd > 1)
    def _():
        w_copy(1).start()

    f_all.wait()

    def body(k, _):
        d = sp_ref[4, k]
        need = sp_ref[3, k]

        @pl.when(need == 1)
        def _():
            w_copy(d).wait()

        @pl.when((need == 1) & (d + 2 < nd))
        def _():
            w_copy(d + 2).start()

        p = sp_ref[2, k]
        c = sp_ref[1, k]
        f_res[p] = _expert_apply(f_res[p], w_buf[d % _NS], b_ref[c])
        return 0

    jax.lax.fori_loop(0, nb, body, 0)
    o_all = pltpu.make_async_copy(f_res, out_hbm, osem)
    o_all.start()
    o_all.wait()


def _big_branch(sp, feat, W, b):
    B, KP, C = feat.shape
    CATE = W.shape[0]
    b3 = b.reshape(CATE, KP, 1)
    grid_spec = pltpu.PrefetchScalarGridSpec(
        num_scalar_prefetch=1,
        grid=(),
        in_specs=[
            pl.BlockSpec(memory_space=pltpu.MemorySpace.HBM),
            pl.BlockSpec(memory_space=pltpu.MemorySpace.HBM),
            pl.BlockSpec((CATE, KP, 1), lambda sp_r: (0, 0, 0)),
        ],
        out_specs=pl.BlockSpec(memory_space=pltpu.MemorySpace.HBM),
        scratch_shapes=[
            pltpu.VMEM((_NS, KP, KP), jnp.float32),
            pltpu.VMEM((B, KP, C), jnp.float32),
            pltpu.SemaphoreType.DMA((_NS,)),
            pltpu.SemaphoreType.DMA,
            pltpu.SemaphoreType.DMA,
        ],
    )
    return pl.pallas_call(
        _big_kernel,
        grid_spec=grid_spec,
        out_shape=jax.ShapeDtypeStruct((B, KP, C), jnp.float32),
    )(sp, feat, W, b3)


def kernel(feat1, feat2, feat3, cls_score, W1, b1, W2, b2, W3, b3):
    B, CATE = cls_score.shape
    KP2 = feat2.shape[1]
    KP3 = feat3.shape[1]
    sp = pl.pallas_call(
        _route_kernel,
        out_shape=jax.ShapeDtypeStruct((8, B), jnp.int32),
    )(cls_score)
    nblk = lambda *shape: pl.BlockSpec(shape, lambda sp_r: (0,) * len(shape))
    key_feat2, key_feat3 = pl.pallas_call(
        _small_kernel,
        grid_spec=pltpu.PrefetchScalarGridSpec(
            num_scalar_prefetch=1,
            grid=(),
            in_specs=[
                nblk(*feat2.shape), nblk(*W2.shape), nblk(CATE, KP2, 1),
                nblk(*feat3.shape), nblk(*W3.shape), nblk(CATE, KP3, 1),
            ],
            out_specs=[nblk(*feat2.shape), nblk(*feat3.shape)],
        ),
        out_shape=(
            jax.ShapeDtypeStruct(feat2.shape, jnp.float32),
            jax.ShapeDtypeStruct(feat3.shape, jnp.float32),
        ),
    )(sp, feat2, W2, b2.reshape(CATE, KP2, 1),
      feat3, W3, b3.reshape(CATE, KP3, 1))
    key_feat1 = _big_branch(sp, feat1, W1, b1)
    return (key_feat1, key_feat2, key_feat3, cls_score)


# speedup vs baseline: 1.4252x; 1.4252x over previous
import jax
import jax.numpy as jnp
from jax.experimental import pallas as pl
from jax.experimental.pallas import tpu as pltpu

_NS = 4
_NQ = 4


def _expert_apply(f, w, brow):
    off = jnp.dot(
        w.astype(jnp.bfloat16),
        f.astype(jnp.bfloat16),
        preferred_element_type=jnp.float32,
    )
    ones = jnp.ones((1, f.shape[1]), jnp.float32)
    bcast = jax.lax.dot_general(brow, ones, (((0,), (0,)), ((), ())),
                                preferred_element_type=jnp.float32)
    off = jnp.maximum(off + bcast, 0.0)
    key = f + off
    mx = jnp.max(key, axis=-1, keepdims=True)
    e = jnp.exp(key - mx)
    return e / jnp.sum(e, axis=-1, keepdims=True)


def _route_kernel(cls_ref, out_ref):
    x = cls_ref[...]
    nb = x.shape[0]
    m = jnp.max(x, axis=-1, keepdims=True)
    iota = jax.lax.broadcasted_iota(jnp.int32, x.shape, 1)
    big = jnp.int32(x.shape[1])
    idx = jnp.min(jnp.where(x == m, iota, big), axis=-1)
    ii = jax.lax.broadcasted_iota(jnp.int32, (nb, nb), 0)
    jj = jax.lax.broadcasted_iota(jnp.int32, (nb, nb), 1)
    cat_row = jnp.broadcast_to(idx[None, :], (nb, nb))
    cat_col = jnp.broadcast_to(idx[:, None], (nb, nb))
    less = (cat_row < cat_col) | ((cat_row == cat_col) & (jj < ii))
    rank = jnp.sum(less.astype(jnp.int32), axis=1)
    eqm = jnp.broadcast_to(rank[None, :], (nb, nb)) == ii
    perm = jnp.sum(jnp.where(eqm, jj, 0), axis=1)
    cats = jnp.sum(jnp.where(eqm, cat_row, 0), axis=1)
    cats_row = jnp.broadcast_to(cats[None, :], (nb, nb))
    prev = jnp.sum(jnp.where(jj == ii - 1, cats_row, 0), axis=1)
    kpos = jax.lax.broadcasted_iota(jnp.int32, (nb,), 0)
    need = ((cats != prev) | (kpos == 0)).astype(jnp.int32)
    need_row = jnp.broadcast_to(need[None, :], (nb, nb))
    dcnt = jnp.sum(jnp.where(jj <= ii, need_row, 0), axis=1) - 1
    nd = jnp.max(dcnt) + 1
    sel = (jnp.broadcast_to(dcnt[None, :], (nb, nb)) == ii) & (need_row == 1)
    dcats = jnp.sum(jnp.where(sel, cats_row, 0), axis=1)
    ch = nb // 4
    lastk = jnp.zeros((nb,), jnp.int32)
    for q in range(4):
        mq = jnp.max(jnp.where((kpos >= ch * q) & (kpos < ch * (q + 1)),
                               rank, 0))
        lastk = jnp.where(kpos == q, mq, lastk)
    out_ref[0, :] = lastk
    out_ref[1, :] = cats
    out_ref[2, :] = perm
    out_ref[3, :] = need
    out_ref[4, :] = dcnt
    out_ref[5, :] = dcats
    out_ref[6, :] = jnp.broadcast_to(nd, (nb,))
    out_ref[7, :] = idx


def _mega_kernel(sp_ref, feat_hbm, w_hbm, b_ref,
                 f2_ref, w2_ref, b2_ref, f3_ref, w3_ref, b3_ref,
                 out_hbm, o2_ref, o3_ref,
                 w_buf, f_res, wsem, fsem, osem):
    nb = f_res.shape[0]
    nd = sp_ref[6, 0]

    def w_copy(d):
        c = sp_ref[5, d]
        return pltpu.make_async_copy(w_hbm.at[c], w_buf.at[d % _NS],
                                     wsem.at[d % _NS])

    f_all = pltpu.make_async_copy(feat_hbm, f_res, fsem)
    f_all.start()
    w_copy(0).start()

    @pl.when(nd > 1)
    def _():
        w_copy(1).start()

    @pl.when(nd > 2)
    def _():
        w_copy(2).start()

    f_all.wait()

    def body(k, _):
        d = sp_ref[4, k]
        need = sp_ref[3, k]
        p = sp_ref[2, k]
        c = sp_ref[1, k]

        o2_ref[p] = _expert_apply(f2_ref[p], w2_ref[c], b2_ref[pl.ds(c, 1), :])
        o3_ref[p] = _expert_apply(f3_ref[p], w3_ref[c], b3_ref[pl.ds(c, 1), :])

        @pl.when(need == 1)
        def _():
            w_copy(d).wait()

        @pl.when((need == 1) & (d + 3 < nd))
        def _():
            w_copy(d + 3).start()

        f_res[p] = _expert_apply(f_res[p], w_buf[d % _NS], b_ref[pl.ds(c, 1), :])

        ch = nb // _NQ
        for q in range(_NQ):
            @pl.when(k == sp_ref[0, q])
            def _():
                pltpu.make_async_copy(f_res.at[pl.ds(q * ch, ch)],
                                      out_hbm.at[pl.ds(q * ch, ch)],
                                      osem.at[q]).start()
        return 0

    jax.lax.fori_loop(0, nb, body, 0)
    ch = nb // _NQ
    for q in range(_NQ):
        pltpu.make_async_copy(f_res.at[pl.ds(q * ch, ch)],
                              out_hbm.at[pl.ds(q * ch, ch)],
                              osem.at[q]).wait()


def kernel(feat1, feat2, feat3, cls_score, W1, b1, W2, b2, W3, b3):
    B, CATE = cls_score.shape
    KP1 = feat1.shape[1]
    KP2 = feat2.shape[1]
    KP3 = feat3.shape[1]
    C = feat1.shape[2]
    sp = pl.pallas_call(
        _route_kernel,
        out_shape=jax.ShapeDtypeStruct((8, B), jnp.int32),
    )(cls_score)
    nblk = lambda *shape: pl.BlockSpec(shape, lambda sp_r: (0,) * len(shape))
    hbm = pl.BlockSpec(memory_space=pltpu.MemorySpace.HBM)
    key_feat1, key_feat2, key_feat3 = pl.pallas_call(
        _mega_kernel,
        grid_spec=pltpu.PrefetchScalarGridSpec(
            num_scalar_prefetch=1,
            grid=(),
            in_specs=[
                hbm, hbm, nblk(CATE, KP1),
                nblk(*feat2.shape), nblk(*W2.shape), nblk(CATE, KP2),
                nblk(*feat3.shape), nblk(*W3.shape), nblk(CATE, KP3),
            ],
            out_specs=[hbm, nblk(*feat2.shape), nblk(*feat3.shape)],
            scratch_shapes=[
                pltpu.VMEM((_NS, KP1, KP1), jnp.float32),
                pltpu.VMEM((B, KP1, C), jnp.float32),
                pltpu.SemaphoreType.DMA((_NS,)),
                pltpu.SemaphoreType.DMA,
                pltpu.SemaphoreType.DMA((_NQ,)),
            ],
        ),
        out_shape=(
            jax.ShapeDtypeStruct(feat1.shape, jnp.float32),
            jax.ShapeDtypeStruct(feat2.shape, jnp.float32),
            jax.ShapeDtypeStruct(feat3.shape, jnp.float32),
        ),
    )(sp, feat1, W1, b1, feat2, W2, b2, feat3, W3, b3)
    return (key_feat1, key_feat2, key_feat3, cls_score)


# speedup vs baseline: 1.4782x; 1.0372x over previous
import jax
import jax.numpy as jnp
from jax.experimental import pallas as pl
from jax.experimental.pallas import tpu as pltpu

_NS = 4
_NQ = 4


def _expert_apply(f, w, brow):
    off = jnp.dot(
        w.astype(jnp.bfloat16),
        f.astype(jnp.bfloat16),
        preferred_element_type=jnp.float32,
    )
    ones = jnp.ones((1, f.shape[1]), jnp.float32)
    bcast = jax.lax.dot_general(brow, ones, (((0,), (0,)), ((), ())),
                                preferred_element_type=jnp.float32)
    off = jnp.maximum(off + bcast, 0.0)
    key = f + off
    mx = jnp.max(key, axis=-1, keepdims=True)
    e = jnp.exp(key - mx)
    return e / jnp.sum(e, axis=-1, keepdims=True)


def _route_meta(x):
    nb = x.shape[0]
    m = jnp.max(x, axis=-1, keepdims=True)
    iota = jax.lax.broadcasted_iota(jnp.int32, x.shape, 1)
    big = jnp.int32(x.shape[1])
    idx = jnp.min(jnp.where(x == m, iota, big), axis=-1)
    ii = jax.lax.broadcasted_iota(jnp.int32, (nb, nb), 0)
    jj = jax.lax.broadcasted_iota(jnp.int32, (nb, nb), 1)
    cat_row = jnp.broadcast_to(idx[None, :], (nb, nb))
    cat_col = jnp.broadcast_to(idx[:, None], (nb, nb))
    less = (cat_row < cat_col) | ((cat_row == cat_col) & (jj < ii))
    rank = jnp.sum(less.astype(jnp.int32), axis=1)
    eqm = jnp.broadcast_to(rank[None, :], (nb, nb)) == ii
    perm = jnp.sum(jnp.where(eqm, jj, 0), axis=1)
    cats = jnp.sum(jnp.where(eqm, cat_row, 0), axis=1)
    cats_row = jnp.broadcast_to(cats[None, :], (nb, nb))
    prev = jnp.sum(jnp.where(jj == ii - 1, cats_row, 0), axis=1)
    kpos = jax.lax.broadcasted_iota(jnp.int32, (nb,), 0)
    need = ((cats != prev) | (kpos == 0)).astype(jnp.int32)
    need_row = jnp.broadcast_to(need[None, :], (nb, nb))
    dcnt = jnp.sum(jnp.where(jj <= ii, need_row, 0), axis=1) - 1
    nd = jnp.max(dcnt) + 1
    sel = (jnp.broadcast_to(dcnt[None, :], (nb, nb)) == ii) & (need_row == 1)
    dcats = jnp.sum(jnp.where(sel, cats_row, 0), axis=1)
    ch = nb // 4
    lastk = jnp.zeros((nb,), jnp.int32)
    for q in range(4):
        mq = jnp.max(jnp.where((kpos >= ch * q) & (kpos < ch * (q + 1)),
                               rank, 0))
        lastk = jnp.where(kpos == q, mq, lastk)
    ndv = jnp.broadcast_to(nd, (nb,))
    rows = [lastk, cats, perm, need, dcnt, dcats, ndv, idx]
    return jnp.concatenate([r[None, :] for r in rows], axis=0)


def _mega_kernel(cls_ref, feat_hbm, w_hbm, b_ref,
                 f2_ref, w2_ref, b2_ref, f3_ref, w3_ref, b3_ref,
                 out_hbm, o2_ref, o3_ref,
                 w_buf, f_res, rt_v, sp_ref, wsem, fsem, osem, rsem):
    nb = f_res.shape[0]
    rt_v[...] = _route_meta(cls_ref[...])
    r_copy = pltpu.make_async_copy(rt_v, sp_ref, rsem)
    r_copy.start()
    r_copy.wait()
    nd = sp_ref[6, 0]

    def w_copy(d):
        c = sp_ref[5, d]
        return pltpu.make_async_copy(w_hbm.at[c], w_buf.at[d % _NS],
                                     wsem.at[d % _NS])

    f_all = pltpu.make_async_copy(feat_hbm, f_res, fsem)
    f_all.start()
    w_copy(0).start()

    @pl.when(nd > 1)
    def _():
        w_copy(1).start()

    @pl.when(nd > 2)
    def _():
        w_copy(2).start()

    f_all.wait()

    def body(k, _):
        d = sp_ref[4, k]
        need = sp_ref[3, k]
        p = sp_ref[2, k]
        c = sp_ref[1, k]

        o2_ref[p] = _expert_apply(f2_ref[p], w2_ref[c], b2_ref[pl.ds(c, 1), :])
        o3_ref[p] = _expert_apply(f3_ref[p], w3_ref[c], b3_ref[pl.ds(c, 1), :])

        @pl.when(need == 1)
        def _():
            w_copy(d).wait()

        @pl.when((need == 1) & (d + 3 < nd))
        def _():
            w_copy(d + 3).start()

        f_res[p] = _expert_apply(f_res[p], w_buf[d % _NS], b_ref[pl.ds(c, 1), :])

        ch = nb // _NQ
        for q in range(_NQ):
            @pl.when(k == sp_ref[0, q])
            def _():
                pltpu.make_async_copy(f_res.at[pl.ds(q * ch, ch)],
                                      out_hbm.at[pl.ds(q * ch, ch)],
                                      osem.at[q]).start()
        return 0

    jax.lax.fori_loop(0, nb, body, 0)
    ch = nb // _NQ
    for q in range(_NQ):
        pltpu.make_async_copy(f_res.at[pl.ds(q * ch, ch)],
                              out_hbm.at[pl.ds(q * ch, ch)],
                              osem.at[q]).wait()


def kernel(feat1, feat2, feat3, cls_score, W1, b1, W2, b2, W3, b3):
    B, CATE = cls_score.shape
    KP1 = feat1.shape[1]
    KP2 = feat2.shape[1]
    KP3 = feat3.shape[1]
    C = feat1.shape[2]
    nblk = lambda *shape: pl.BlockSpec(shape, lambda: (0,) * len(shape))
    hbm = pl.BlockSpec(memory_space=pltpu.MemorySpace.HBM)
    key_feat1, key_feat2, key_feat3 = pl.pallas_call(
        _mega_kernel,
        in_specs=[
            nblk(B, CATE), hbm, hbm, nblk(CATE, KP1),
            nblk(*feat2.shape), nblk(*W2.shape), nblk(CATE, KP2),
            nblk(*feat3.shape), nblk(*W3.shape), nblk(CATE, KP3),
        ],
        out_specs=[hbm, nblk(*feat2.shape), nblk(*feat3.shape)],
        scratch_shapes=[
            pltpu.VMEM((_NS, KP1, KP1), jnp.float32),
            pltpu.VMEM((B, KP1, C), jnp.float32),
            pltpu.VMEM((8, B), jnp.int32),
            pltpu.SMEM((8, B), jnp.int32),
            pltpu.SemaphoreType.DMA((_NS,)),
            pltpu.SemaphoreType.DMA,
            pltpu.SemaphoreType.DMA((_NQ,)),
            pltpu.SemaphoreType.DMA,
        ],
        out_shape=(
            jax.ShapeDtypeStruct(feat1.shape, jnp.float32),
            jax.ShapeDtypeStruct(feat2.shape, jnp.float32),
            jax.ShapeDtypeStruct(feat3.shape, jnp.float32),
        ),
    )(cls_score, feat1, W1, b1, feat2, W2, b2, feat3, W3, b3)
    return (key_feat1, key_feat2, key_feat3, cls_score)


# speedup vs baseline: 1.4833x; 1.0034x over previous
import jax
import jax.numpy as jnp
from jax.experimental import pallas as pl
from jax.experimental.pallas import tpu as pltpu

_NS = 4
_NQ = 4


def _expert_apply(f, w, brow):
    off = jnp.dot(
        w.astype(jnp.bfloat16),
        f.astype(jnp.bfloat16),
        preferred_element_type=jnp.float32,
    )
    ones = jnp.ones((1, f.shape[1]), jnp.float32)
    bcast = jax.lax.dot_general(brow, ones, (((0,), (0,)), ((), ())),
                                preferred_element_type=jnp.float32)
    off = jnp.maximum(off + bcast, 0.0)
    key = f + off
    mx = jnp.max(key, axis=-1, keepdims=True)
    e = jnp.exp(key - mx)
    return e / jnp.sum(e, axis=-1, keepdims=True)


def _route_meta(x):
    nb = x.shape[0]
    m = jnp.max(x, axis=-1, keepdims=True)
    iota = jax.lax.broadcasted_iota(jnp.int32, x.shape, 1)
    big = jnp.int32(x.shape[1])
    idx = jnp.min(jnp.where(x == m, iota, big), axis=-1)
    ii = jax.lax.broadcasted_iota(jnp.int32, (nb, nb), 0)
    jj = jax.lax.broadcasted_iota(jnp.int32, (nb, nb), 1)
    cat_row = jnp.broadcast_to(idx[None, :], (nb, nb))
    cat_col = jnp.broadcast_to(idx[:, None], (nb, nb))
    less = (cat_row < cat_col) | ((cat_row == cat_col) & (jj < ii))
    rank = jnp.sum(less.astype(jnp.int32), axis=1)
    eqm = jnp.broadcast_to(rank[None, :], (nb, nb)) == ii
    perm = jnp.sum(jnp.where(eqm, jj, 0), axis=1)
    cats = jnp.sum(jnp.where(eqm, cat_row, 0), axis=1)
    cats_row = jnp.broadcast_to(cats[None, :], (nb, nb))
    prev = jnp.sum(jnp.where(jj == ii - 1, cats_row, 0), axis=1)
    kpos = jax.lax.broadcasted_iota(jnp.int32, (nb,), 0)
    need = ((cats != prev) | (kpos == 0)).astype(jnp.int32)
    need_row = jnp.broadcast_to(need[None, :], (nb, nb))
    dcnt = jnp.sum(jnp.where(jj <= ii, need_row, 0), axis=1) - 1
    nd = jnp.max(dcnt) + 1
    sel = (jnp.broadcast_to(dcnt[None, :], (nb, nb)) == ii) & (need_row == 1)
    dcats = jnp.sum(jnp.where(sel, cats_row, 0), axis=1)
    ch = nb // 4
    lastk = jnp.zeros((nb,), jnp.int32)
    for q in range(4):
        mq = jnp.max(jnp.where((kpos >= ch * q) & (kpos < ch * (q + 1)),
                               rank, 0))
        lastk = jnp.where(kpos == q, mq, lastk)
    ndv = jnp.broadcast_to(nd, (nb,))
    rows = [lastk, cats, perm, need, dcnt, dcats, ndv, idx]
    return jnp.concatenate([r[None, :] for r in rows], axis=0)


def _mega_kernel(cls_ref, feat_hbm, w_hbm, b_ref,
                 f2_ref, w2_ref, b2_ref, f3_ref, w3_ref, b3_ref,
                 out_hbm, o2_ref, o3_ref,
                 w_buf, f_res, rt_v, sp_ref, wsem, fsem, osem, rsem):
    nb = f_res.shape[0]
    rt_v[...] = _route_meta(cls_ref[...])
    r_copy = pltpu.make_async_copy(rt_v, sp_ref, rsem)
    r_copy.start()
    r_copy.wait()
    nd = sp_ref[6, 0]

    def w_copy(d):
        c = sp_ref[5, d]
        return pltpu.make_async_copy(w_hbm.at[c], w_buf.at[d % _NS],
                                     wsem.at[d % _NS])

    f_all = pltpu.make_async_copy(feat_hbm, f_res, fsem)
    f_all.start()
    w_copy(0).start()

    @pl.when(nd > 1)
    def _():
        w_copy(1).start()

    @pl.when(nd > 2)
    def _():
        w_copy(2).start()


    def body(k, _):
        d = sp_ref[4, k]
        need = sp_ref[3, k]
        p = sp_ref[2, k]
        c = sp_ref[1, k]

        o2_ref[p] = _expert_apply(f2_ref[p], w2_ref[c], b2_ref[pl.ds(c, 1), :])
        o3_ref[p] = _expert_apply(f3_ref[p], w3_ref[c], b3_ref[pl.ds(c, 1), :])

        @pl.when(need == 1)
        def _():
            w_copy(d).wait()

        @pl.when((need == 1) & (d + 3 < nd))
        def _():
            w_copy(d + 3).start()

        @pl.when(k == 0)
        def _():
            f_all.wait()

        f_res[p] = _expert_apply(f_res[p], w_buf[d % _NS], b_ref[pl.ds(c, 1), :])

        ch = nb // _NQ
        for q in range(_NQ):
            @pl.when(k == sp_ref[0, q])
            def _():
                pltpu.make_async_copy(f_res.at[pl.ds(q * ch, ch)],
                                      out_hbm.at[pl.ds(q * ch, ch)],
                                      osem.at[q]).start()
        return 0

    jax.lax.fori_loop(0, nb, body, 0)
    ch = nb // _NQ
    for q in range(_NQ):
        pltpu.make_async_copy(f_res.at[pl.ds(q * ch, ch)],
                              out_hbm.at[pl.ds(q * ch, ch)],
                              osem.at[q]).wait()


def kernel(feat1, feat2, feat3, cls_score, W1, b1, W2, b2, W3, b3):
    B, CATE = cls_score.shape
    KP1 = feat1.shape[1]
    KP2 = feat2.shape[1]
    KP3 = feat3.shape[1]
    C = feat1.shape[2]
    nblk = lambda *shape: pl.BlockSpec(shape, lambda: (0,) * len(shape))
    hbm = pl.BlockSpec(memory_space=pltpu.MemorySpace.HBM)
    key_feat1, key_feat2, key_feat3 = pl.pallas_call(
        _mega_kernel,
        in_specs=[
            nblk(B, CATE), hbm, hbm, nblk(CATE, KP1),
            nblk(*feat2.shape), nblk(*W2.shape), nblk(CATE, KP2),
            nblk(*feat3.shape), nblk(*W3.shape), nblk(CATE, KP3),
        ],
        out_specs=[hbm, nblk(*feat2.shape), nblk(*feat3.shape)],
        scratch_shapes=[
            pltpu.VMEM((_NS, KP1, KP1), jnp.float32),
            pltpu.VMEM((B, KP1, C), jnp.float32),
            pltpu.VMEM((8, B), jnp.int32),
            pltpu.SMEM((8, B), jnp.int32),
            pltpu.SemaphoreType.DMA((_NS,)),
            pltpu.SemaphoreType.DMA,
            pltpu.SemaphoreType.DMA((_NQ,)),
            pltpu.SemaphoreType.DMA,
        ],
        out_shape=(
            jax.ShapeDtypeStruct(feat1.shape, jnp.float32),
            jax.ShapeDtypeStruct(feat2.shape, jnp.float32),
            jax.ShapeDtypeStruct(feat3.shape, jnp.float32),
        ),
    )(cls_score, feat1, W1, b1, feat2, W2, b2, feat3, W3, b3)
    return (key_feat1, key_feat2, key_feat3, cls_score)


# speedup vs baseline: 1.5343x; 1.0344x over previous
import jax
import jax.numpy as jnp
from jax.experimental import pallas as pl
from jax.experimental.pallas import tpu as pltpu

_NS = 5
_NQ = 4


def _expert_apply(f, w, brow):
    off = jnp.dot(
        w.astype(jnp.bfloat16),
        f.astype(jnp.bfloat16),
        preferred_element_type=jnp.float32,
    )
    ones = jnp.ones((1, f.shape[1]), jnp.float32)
    bcast = jax.lax.dot_general(brow, ones, (((0,), (0,)), ((), ())),
                                preferred_element_type=jnp.float32)
    off = jnp.maximum(off + bcast, 0.0)
    key = f + off
    mx = jnp.max(key, axis=-1, keepdims=True)
    e = jnp.exp(key - mx)
    return e / jnp.sum(e, axis=-1, keepdims=True)


def _route_meta(x):
    nb = x.shape[0]
    m = jnp.max(x, axis=-1, keepdims=True)
    iota = jax.lax.broadcasted_iota(jnp.int32, x.shape, 1)
    big = jnp.int32(x.shape[1])
    idx = jnp.min(jnp.where(x == m, iota, big), axis=-1)
    ii = jax.lax.broadcasted_iota(jnp.int32, (nb, nb), 0)
    jj = jax.lax.broadcasted_iota(jnp.int32, (nb, nb), 1)
    cat_row = jnp.broadcast_to(idx[None, :], (nb, nb))
    cat_col = jnp.broadcast_to(idx[:, None], (nb, nb))
    less = (cat_row < cat_col) | ((cat_row == cat_col) & (jj < ii))
    rank = jnp.sum(less.astype(jnp.int32), axis=1)
    eqm = jnp.broadcast_to(rank[None, :], (nb, nb)) == ii
    perm = jnp.sum(jnp.where(eqm, jj, 0), axis=1)
    cats = jnp.sum(jnp.where(eqm, cat_row, 0), axis=1)
    cats_row = jnp.broadcast_to(cats[None, :], (nb, nb))
    prev = jnp.sum(jnp.where(jj == ii - 1, cats_row, 0), axis=1)
    kpos = jax.lax.broadcasted_iota(jnp.int32, (nb,), 0)
    need = ((cats != prev) | (kpos == 0)).astype(jnp.int32)
    need_row = jnp.broadcast_to(need[None, :], (nb, nb))
    dcnt = jnp.sum(jnp.where(jj <= ii, need_row, 0), axis=1) - 1
    nd = jnp.max(dcnt) + 1
    sel = (jnp.broadcast_to(dcnt[None, :], (nb, nb)) == ii) & (need_row == 1)
    dcats = jnp.sum(jnp.where(sel, cats_row, 0), axis=1)
    ch = nb // 4
    lastk = jnp.zeros((nb,), jnp.int32)
    for q in range(4):
        mq = jnp.max(jnp.where((kpos >= ch * q) & (kpos < ch * (q + 1)),
                               rank, 0))
        lastk = jnp.where(kpos == q, mq, lastk)
    ndv = jnp.broadcast_to(nd, (nb,))
    rows = [lastk, cats, perm, need, dcnt, dcats, ndv, idx]
    return jnp.concatenate([r[None, :] for r in rows], axis=0)


def _mega_kernel(cls_ref, feat_hbm, w_hbm, b_ref,
                 f2_ref, w2_hbm, b2_ref, f3_ref, w3_ref, b3_ref,
                 out_hbm, o2_ref, o3_ref,
                 w_buf, w2_buf, f_res, rt_v, sp_ref,
                 wsem, wsem2, fsem, osem, rsem):
    nb = f_res.shape[0]
    rt_v[...] = _route_meta(cls_ref[...])
    r_copy = pltpu.make_async_copy(rt_v, sp_ref, rsem)
    r_copy.start()
    r_copy.wait()
    nd = sp_ref[6, 0]

    def w_copy(d):
        c = sp_ref[5, d]
        return pltpu.make_async_copy(w_hbm.at[c], w_buf.at[d % _NS],
                                     wsem.at[d % _NS])

    def w2_copy(d):
        c = sp_ref[5, d]
        return pltpu.make_async_copy(w2_hbm.at[c], w2_buf.at[d % _NS],
                                     wsem2.at[d % _NS])

    f_all = pltpu.make_async_copy(feat_hbm, f_res, fsem)
    f_all.start()
    w2_copy(0).start()
    w_copy(0).start()

    for _i in (1, 2, 3):
        @pl.when(nd > _i)
        def _(i=_i):
            w2_copy(i).start()
            w_copy(i).start()


    def body(k, _):
        d = sp_ref[4, k]
        need = sp_ref[3, k]
        p = sp_ref[2, k]
        c = sp_ref[1, k]

        @pl.when(need == 1)
        def _():
            w2_copy(d).wait()

        o2_ref[p] = _expert_apply(f2_ref[p], w2_buf[d % _NS],
                                  b2_ref[pl.ds(c, 1), :])
        o3_ref[p] = _expert_apply(f3_ref[p], w3_ref[c], b3_ref[pl.ds(c, 1), :])

        @pl.when(need == 1)
        def _():
            w_copy(d).wait()

        @pl.when((need == 1) & (d + 4 < nd))
        def _():
            w2_copy(d + 4).start()
            w_copy(d + 4).start()

        @pl.when(k == 0)
        def _():
            f_all.wait()

        f_res[p] = _expert_apply(f_res[p], w_buf[d % _NS], b_ref[pl.ds(c, 1), :])

        ch = nb // _NQ
        for q in range(_NQ):
            @pl.when(k == sp_ref[0, q])
            def _():
                pltpu.make_async_copy(f_res.at[pl.ds(q * ch, ch)],
                                      out_hbm.at[pl.ds(q * ch, ch)],
                                      osem.at[q]).start()
        return 0

    jax.lax.fori_loop(0, nb, body, 0)
    ch = nb // _NQ
    for q in range(_NQ):
        pltpu.make_async_copy(f_res.at[pl.ds(q * ch, ch)],
                              out_hbm.at[pl.ds(q * ch, ch)],
                              osem.at[q]).wait()


def kernel(feat1, feat2, feat3, cls_score, W1, b1, W2, b2, W3, b3):
    B, CATE = cls_score.shape
    KP1 = feat1.shape[1]
    KP2 = feat2.shape[1]
    KP3 = feat3.shape[1]
    C = feat1.shape[2]
    nblk = lambda *shape: pl.BlockSpec(shape, lambda: (0,) * len(shape))
    hbm = pl.BlockSpec(memory_space=pltpu.MemorySpace.HBM)
    key_feat1, key_feat2, key_feat3 = pl.pallas_call(
        _mega_kernel,
        in_specs=[
            nblk(B, CATE), hbm, hbm, nblk(CATE, KP1),
            nblk(*feat2.shape), hbm, nblk(CATE, KP2),
            nblk(*feat3.shape), nblk(*W3.shape), nblk(CATE, KP3),
        ],
        out_specs=[hbm, nblk(*feat2.shape), nblk(*feat3.shape)],
        scratch_shapes=[
            pltpu.VMEM((_NS, KP1, KP1), jnp.float32),
            pltpu.VMEM((_NS, KP2, KP2), jnp.float32),
            pltpu.VMEM((B, KP1, C), jnp.float32),
            pltpu.VMEM((8, B), jnp.int32),
            pltpu.SMEM((8, B), jnp.int32),
            pltpu.SemaphoreType.DMA((_NS,)),
            pltpu.SemaphoreType.DMA((_NS,)),
            pltpu.SemaphoreType.DMA,
            pltpu.SemaphoreType.DMA((_NQ,)),
            pltpu.SemaphoreType.DMA,
        ],
        out_shape=(
            jax.ShapeDtypeStruct(feat1.shape, jnp.float32),
            jax.ShapeDtypeStruct(feat2.shape, jnp.float32),
            jax.ShapeDtypeStruct(feat3.shape, jnp.float32),
        ),
    )(cls_score, feat1, W1, b1, feat2, W2, b2, feat3, W3, b3)
    return (key_feat1, key_feat2, key_feat3, cls_score)
